# Initial kernel scaffold; baseline (speedup 1.0000x reference)
import jax
import jax.numpy as jnp
from jax.experimental import pallas as pl


def kernel(hidden_states, wg, w1, b1, w2, b2):
    raise NotImplementedError("write your pallas kernel here")


# speedup vs baseline: 1.0751x; 1.0751x over previous
import functools

import jax
import jax.numpy as jnp
from jax import lax
from jax.experimental import pallas as pl
from jax.experimental.pallas import tpu as pltpu
from jax.experimental.pallas import tpu_sc as plsc

S = 4096
D = 1024
E = 64
DFF = 2048
CAP = 64
TB = 256
NB = S // TB
NSLOT = E * CAP
TRASH = NSLOT
NDISP = NSLOT + CAP
GW = 128
NW = 32
TPW = S // NW
HALF = TPW // 2


def _routing_body(x_ref, wg_ref, t2s_ref, gate_ref, stats_ref, acc_ref):
    step = pl.program_id(0)

    @pl.when(step == 0)
    def _init():
        acc_ref[...] = jnp.zeros_like(acc_ref)

    x = x_ref[0]
    logits = jnp.dot(x, wg_ref[...], preferred_element_type=jnp.float32)
    m = jnp.max(logits, axis=1, keepdims=True)
    ex = jnp.exp(logits - m)
    gates = ex / jnp.sum(ex, axis=1, keepdims=True)

    iota_e = lax.broadcasted_iota(jnp.int32, (TB, E), 1)
    eq = logits == m
    idxe = jnp.min(jnp.where(eq, iota_e, E), axis=1)
    maskf = (iota_e == idxe[:, None]).astype(jnp.float32)

    ti = lax.broadcasted_iota(jnp.int32, (TB, TB), 0)
    tj = lax.broadcasted_iota(jnp.int32, (TB, TB), 1)
    tri = (tj < ti).astype(jnp.float32)
    carry = acc_ref[0:1, 0:E]
    pos = jnp.dot(tri, maskf, preferred_element_type=jnp.float32) + carry
    keptf = maskf * (pos < CAP).astype(jnp.float32)

    gate_s = jnp.sum(gates * keptf, axis=1)
    pos_s = jnp.sum(pos * keptf, axis=1)
    has = jnp.sum(keptf, axis=1) > 0.0
    slot = jnp.where(has, idxe * CAP + pos_s.astype(jnp.int32), TRASH)
    t2s_ref[0, 0, :] = slot
    gate_ref[0, 0, :] = gate_s

    acc_ref[0:1, 0:E] = carry + jnp.sum(maskf, axis=0, keepdims=True)
    acc_ref[1:2, 0:E] = acc_ref[1:2, 0:E] + jnp.sum(gates, axis=0, keepdims=True)

    @pl.when(step == NB - 1)
    def _fin():
        counts = acc_ref[0:1, 0:E]
        mes = acc_ref[1:2, 0:E]
        l_aux = jnp.sum((mes / S) * (counts / S)) * E
        stats_ref[0:1, 0:E] = counts
        stats_ref[1:2, 0:1] = jnp.reshape(l_aux, (1, 1))


_routing = pl.pallas_call(
    _routing_body,
    grid=(NB,),
    in_specs=[
        pl.BlockSpec((1, TB, D), lambda i: (i, 0, 0)),
        pl.BlockSpec((D, E), lambda i: (0, 0)),
    ],
    out_specs=[
        pl.BlockSpec((1, 1, TB), lambda i: (i, 0, 0)),
        pl.BlockSpec((1, 1, TB), lambda i: (i, 0, 0)),
        pl.BlockSpec((8, 128), lambda i: (0, 0)),
    ],
    out_shape=[
        jax.ShapeDtypeStruct((NB, 1, TB), jnp.int32),
        jax.ShapeDtypeStruct((NB, 1, TB), jnp.float32),
        jax.ShapeDtypeStruct((8, 128), jnp.float32),
    ],
    scratch_shapes=[pltpu.VMEM((8, 128), jnp.float32)],
)


@functools.cache
def _make_dispatch():
    mesh = plsc.VectorSubcoreMesh(core_axis_name="c", subcore_axis_name="s")

    @functools.partial(
        pl.kernel,
        out_type=[
            jax.ShapeDtypeStruct((NDISP, D), jnp.float32),
            jax.ShapeDtypeStruct((NDISP, GW), jnp.float32),
        ],
        mesh=mesh,
        scratch_types=[
            pltpu.VMEM((HALF,), jnp.int32),
            pltpu.VMEM((HALF,), jnp.int32),
            pltpu.VMEM((HALF, D), jnp.float32),
            pltpu.VMEM((HALF, GW), jnp.float32),
            pltpu.VMEM((HALF, GW), jnp.float32),
            pltpu.SemaphoreType.DMA,
            pltpu.SemaphoreType.DMA,
        ],
    )
    def _dispatch(x_hbm, idx3_hbm, g3_hbm, disp_hbm, sg_hbm,
                  idx_a, idx_b, rows_v, g_a, g_b, sem, gsem):
        wid = lax.axis_index("s") * 2 + lax.axis_index("c")
        base = wid * TPW
        pltpu.sync_copy(idx3_hbm.at[wid, 0], idx_a)
        pltpu.sync_copy(idx3_hbm.at[wid, 1], idx_b)
        pltpu.sync_copy(g3_hbm.at[wid, 0], g_a)
        pltpu.sync_copy(g3_hbm.at[wid, 1], g_b)
        ga_dma = pltpu.async_copy(g_a, sg_hbm.at[idx_a], gsem)
        gb_dma = pltpu.async_copy(g_b, sg_hbm.at[idx_b], gsem)
        pltpu.sync_copy(x_hbm.at[pl.ds(base, HALF)], rows_v)
        pltpu.async_copy(rows_v, disp_hbm.at[idx_a], sem).wait()
        pltpu.sync_copy(x_hbm.at[pl.ds(base + HALF, HALF)], rows_v)
        pltpu.async_copy(rows_v, disp_hbm.at[idx_b], sem).wait()
        ga_dma.wait()
        gb_dma.wait()

    return _dispatch


def _experts_body(cnt_ref, disp_ref, sg_ref, w1_ref, b1_ref, w2_ref, b2_ref,
                  out_ref):
    e = pl.program_id(0)

    @pl.when(e < E)
    def _mlp():
        cnt = cnt_ref[jnp.minimum(e, E - 1)]
        rows = lax.broadcasted_iota(jnp.int32, (CAP, D), 0)
        keep = rows < cnt
        xb = jnp.where(keep, disp_ref[...], 0.0)
        h = jnp.dot(xb, w1_ref[0], preferred_element_type=jnp.float32)
        h = jnp.maximum(h + b1_ref[0], 0.0)
        y = jnp.dot(h, w2_ref[0], preferred_element_type=jnp.float32)
        y = y + b2_ref[0]
        sg = sg_ref[:, 0:1]
        out_ref[...] = jnp.where(keep, y * sg, 0.0)

    @pl.when(e >= E)
    def _trash():
        out_ref[...] = jnp.zeros_like(out_ref)


_experts = pl.pallas_call(
    _experts_body,
    grid=(E + 1,),
    in_specs=[
        pl.BlockSpec(memory_space=pltpu.SMEM),
        pl.BlockSpec((CAP, D), lambda e: (e, 0)),
        pl.BlockSpec((CAP, GW), lambda e: (e, 0)),
        pl.BlockSpec((1, D, DFF), lambda e: (jnp.minimum(e, E - 1), 0, 0)),
        pl.BlockSpec((1, 1, DFF), lambda e: (jnp.minimum(e, E - 1), 0, 0)),
        pl.BlockSpec((1, DFF, D), lambda e: (jnp.minimum(e, E - 1), 0, 0)),
        pl.BlockSpec((1, 1, D), lambda e: (jnp.minimum(e, E - 1), 0, 0)),
    ],
    out_specs=pl.BlockSpec((CAP, D), lambda e: (e, 0)),
    out_shape=jax.ShapeDtypeStruct((NDISP, D), jnp.float32),
)


@functools.cache
def _make_combine():
    mesh = plsc.VectorSubcoreMesh(core_axis_name="c", subcore_axis_name="s")

    @functools.partial(
        pl.kernel,
        out_type=jax.ShapeDtypeStruct((S, D), jnp.float32),
        mesh=mesh,
        scratch_types=[
            pltpu.VMEM((HALF,), jnp.int32),
            pltpu.VMEM((HALF,), jnp.int32),
            pltpu.VMEM((HALF, D), jnp.float32),
            pltpu.SemaphoreType.DMA,
        ],
    )
    def _combine(eo_hbm, idx3_hbm, out_hbm, idx_a, idx_b, rows_v, sem):
        wid = lax.axis_index("s") * 2 + lax.axis_index("c")
        base = wid * TPW
        pltpu.sync_copy(idx3_hbm.at[wid, 0], idx_a)
        pltpu.sync_copy(idx3_hbm.at[wid, 1], idx_b)
        pltpu.async_copy(eo_hbm.at[idx_a], rows_v, sem).wait()
        pltpu.sync_copy(rows_v, out_hbm.at[pl.ds(base, HALF)])
        pltpu.async_copy(eo_hbm.at[idx_b], rows_v, sem).wait()
        pltpu.sync_copy(rows_v, out_hbm.at[pl.ds(base + HALF, HALF)])

    return _combine


def kernel(hidden_states, wg, w1, b1, w2, b2):
    x = hidden_states.reshape(S, D)
    t2s3, gate3, stats = _routing(x.reshape(NB, TB, D), wg)
    exp_counts = stats[0, :E].astype(jnp.int32)
    l_aux = stats[1, 0]
    idx3 = t2s3.reshape(NW, 2, HALF)
    g3 = jnp.broadcast_to(gate3.reshape(S, 1), (S, GW)).reshape(NW, 2, HALF, GW)
    disp, sg = _make_dispatch()(x, idx3, g3)
    eo = _experts(exp_counts, disp, sg, w1, b1.reshape(E, 1, DFF),
                  w2, b2.reshape(E, 1, D))
    out = _make_combine()(eo, idx3)
    return out.reshape(hidden_states.shape), l_aux, exp_counts


# speedup vs baseline: 1.0862x; 1.0103x over previous
import functools

import jax
import jax.numpy as jnp
from jax import lax
from jax.experimental import pallas as pl
from jax.experimental.pallas import tpu as pltpu
from jax.experimental.pallas import tpu_sc as plsc

S = 4096
D = 1024
E = 64
DFF = 2048
CAP = 64
TB = 256
NB = S // TB
NSLOT = E * CAP
TRASH = NSLOT
NDISP = NSLOT + CAP
GW = 128
NW = 32
TPW = S // NW
NCH = 4
CH = TPW // NCH


def _routing_body(x_ref, wg_ref, t2s_ref, gate_ref, stats_ref, acc_ref):
    step = pl.program_id(0)

    @pl.when(step == 0)
    def _init():
        acc_ref[...] = jnp.zeros_like(acc_ref)

    x = x_ref[0]
    logits = jnp.dot(x, wg_ref[...], preferred_element_type=jnp.float32)
    m = jnp.max(logits, axis=1, keepdims=True)
    ex = jnp.exp(logits - m)
    gates = ex / jnp.sum(ex, axis=1, keepdims=True)

    iota_e = lax.broadcasted_iota(jnp.int32, (TB, E), 1)
    eq = logits == m
    idxe = jnp.min(jnp.where(eq, iota_e, E), axis=1)
    maskf = (iota_e == idxe[:, None]).astype(jnp.float32)

    ti = lax.broadcasted_iota(jnp.int32, (TB, TB), 0)
    tj = lax.broadcasted_iota(jnp.int32, (TB, TB), 1)
    tri = (tj < ti).astype(jnp.float32)
    carry = acc_ref[0:1, 0:E]
    pos = jnp.dot(tri, maskf, preferred_element_type=jnp.float32) + carry
    keptf = maskf * (pos < CAP).astype(jnp.float32)

    gate_s = jnp.sum(gates * keptf, axis=1)
    pos_s = jnp.sum(pos * keptf, axis=1)
    has = jnp.sum(keptf, axis=1) > 0.0
    slot = jnp.where(has, idxe * CAP + pos_s.astype(jnp.int32), TRASH)
    t2s_ref[0, 0, :] = slot
    gate_ref[...] = jnp.broadcast_to(gate_s[:, None], (TB, GW))

    acc_ref[0:1, 0:E] = carry + jnp.sum(maskf, axis=0, keepdims=True)
    acc_ref[1:2, 0:E] = acc_ref[1:2, 0:E] + jnp.sum(gates, axis=0, keepdims=True)

    @pl.when(step == NB - 1)
    def _fin():
        counts = acc_ref[0:1, 0:E]
        mes = acc_ref[1:2, 0:E]
        l_aux = jnp.sum((mes / S) * (counts / S)) * E
        stats_ref[0:1, 0:E] = counts
        stats_ref[1:2, 0:1] = jnp.reshape(l_aux, (1, 1))


_routing = pl.pallas_call(
    _routing_body,
    grid=(NB,),
    in_specs=[
        pl.BlockSpec((1, TB, D), lambda i: (i, 0, 0)),
        pl.BlockSpec((D, E), lambda i: (0, 0)),
    ],
    out_specs=[
        pl.BlockSpec((1, 1, TB), lambda i: (i, 0, 0)),
        pl.BlockSpec((TB, GW), lambda i: (i, 0)),
        pl.BlockSpec((8, 128), lambda i: (0, 0)),
    ],
    out_shape=[
        jax.ShapeDtypeStruct((NB, 1, TB), jnp.int32),
        jax.ShapeDtypeStruct((S, GW), jnp.float32),
        jax.ShapeDtypeStruct((8, 128), jnp.float32),
    ],
    scratch_shapes=[pltpu.VMEM((8, 128), jnp.float32)],
)


@functools.cache
def _make_dispatch():
    mesh = plsc.VectorSubcoreMesh(core_axis_name="c", subcore_axis_name="s")

    @functools.partial(
        pl.kernel,
        out_type=[
            jax.ShapeDtypeStruct((NDISP, D), jnp.float32),
            jax.ShapeDtypeStruct((NDISP, GW), jnp.float32),
        ],
        mesh=mesh,
        scratch_types=[
            pltpu.VMEM((NCH, CH), jnp.int32),
            pltpu.VMEM((CH, D), jnp.float32),
            pltpu.VMEM((CH, D), jnp.float32),
            pltpu.VMEM((NCH, CH, GW), jnp.float32),
            pltpu.SemaphoreType.DMA,
            pltpu.SemaphoreType.DMA,
            pltpu.SemaphoreType.DMA,
            pltpu.SemaphoreType.DMA,
            pltpu.SemaphoreType.DMA,
        ],
    )
    def _dispatch(x_hbm, idx4_hbm, g4_hbm, disp_hbm, sg_hbm,
                  idx_v, b0, b1, g_all, l0, l1, s0, s1, gs):
        wid = lax.axis_index("s") * 2 + lax.axis_index("c")
        base = wid * TPW
        pltpu.sync_copy(idx4_hbm.at[wid], idx_v)
        pltpu.sync_copy(g4_hbm.at[wid], g_all)
        gate_dmas = [
            pltpu.async_copy(g_all.at[q], sg_hbm.at[idx_v.at[q]], gs)
            for q in range(NCH)
        ]
        bufs = (b0, b1)
        lsems = (l0, l1)
        ssems = (s0, s1)
        loads = [None] * NCH
        scats = [None] * NCH

        def load(q):
            b = q % 2
            loads[q] = pltpu.async_copy(
                x_hbm.at[pl.ds(base + q * CH, CH)], bufs[b], lsems[b])

        load(0)
        load(1)
        for q in range(NCH):
            b = q % 2
            loads[q].wait()
            scats[q] = pltpu.async_copy(bufs[b], disp_hbm.at[idx_v.at[q]],
                                        ssems[b])
            if q + 2 < NCH:
                scats[q].wait()
                load(q + 2)
        for q in range(NCH - 2, NCH):
            scats[q].wait()
        for dma in gate_dmas:
            dma.wait()

    return _dispatch


def _experts_body(cnt_ref, disp_ref, sg_ref, w1_ref, b1_ref, w2_ref, b2_ref,
                  out_ref):
    e = pl.program_id(0)

    @pl.when(e < E)
    def _mlp():
        cnt = cnt_ref[jnp.minimum(e, E - 1)]
        rows = lax.broadcasted_iota(jnp.int32, (CAP, D), 0)
        keep = rows < cnt
        xb = jnp.where(keep, disp_ref[...], 0.0)
        h = jnp.dot(xb, w1_ref[0], preferred_element_type=jnp.float32)
        h = jnp.maximum(h + b1_ref[0], 0.0)
        y = jnp.dot(h, w2_ref[0], preferred_element_type=jnp.float32)
        y = y + b2_ref[0]
        sg = sg_ref[:, 0:1]
        out_ref[...] = jnp.where(keep, y * sg, 0.0)

    @pl.when(e >= E)
    def _trash():
        out_ref[...] = jnp.zeros_like(out_ref)


_experts = pl.pallas_call(
    _experts_body,
    grid=(E + 1,),
    in_specs=[
        pl.BlockSpec(memory_space=pltpu.SMEM),
        pl.BlockSpec((CAP, D), lambda e: (e, 0)),
        pl.BlockSpec((CAP, GW), lambda e: (e, 0)),
        pl.BlockSpec((1, D, DFF), lambda e: (jnp.minimum(e, E - 1), 0, 0)),
        pl.BlockSpec((1, 1, DFF), lambda e: (jnp.minimum(e, E - 1), 0, 0)),
        pl.BlockSpec((1, DFF, D), lambda e: (jnp.minimum(e, E - 1), 0, 0)),
        pl.BlockSpec((1, 1, D), lambda e: (jnp.minimum(e, E - 1), 0, 0)),
    ],
    out_specs=pl.BlockSpec((CAP, D), lambda e: (e, 0)),
    out_shape=jax.ShapeDtypeStruct((NDISP, D), jnp.float32),
)


@functools.cache
def _make_combine():
    mesh = plsc.VectorSubcoreMesh(core_axis_name="c", subcore_axis_name="s")

    @functools.partial(
        pl.kernel,
        out_type=jax.ShapeDtypeStruct((S, D), jnp.float32),
        mesh=mesh,
        scratch_types=[
            pltpu.VMEM((NCH, CH), jnp.int32),
            pltpu.VMEM((CH, D), jnp.float32),
            pltpu.VMEM((CH, D), jnp.float32),
            pltpu.SemaphoreType.DMA,
            pltpu.SemaphoreType.DMA,
            pltpu.SemaphoreType.DMA,
            pltpu.SemaphoreType.DMA,
        ],
    )
    def _combine(eo_hbm, idx4_hbm, out_hbm, idx_v, b0, b1, g0, g1, st0, st1):
        wid = lax.axis_index("s") * 2 + lax.axis_index("c")
        base = wid * TPW
        pltpu.sync_copy(idx4_hbm.at[wid], idx_v)
        bufs = (b0, b1)
        gsems = (g0, g1)
        ssems = (st0, st1)
        gets = [None] * NCH
        puts = [None] * NCH
        for q in range(2):
            gets[q] = pltpu.async_copy(eo_hbm.at[idx_v.at[q]], bufs[q],
                                       gsems[q])
        for q in range(NCH):
            b = q % 2
            gets[q].wait()
            puts[q] = pltpu.async_copy(
                bufs[b], out_hbm.at[pl.ds(base + q * CH, CH)], ssems[b])
            if q + 2 < NCH:
                nq = q + 2
                puts[q].wait()
                gets[nq] = pltpu.async_copy(eo_hbm.at[idx_v.at[nq]], bufs[b],
                                            gsems[b])
        for q in range(NCH - 2, NCH):
            puts[q].wait()

    return _combine


def kernel(hidden_states, wg, w1, b1, w2, b2):
    x = hidden_states.reshape(S, D)
    t2s3, gtab, stats = _routing(x.reshape(NB, TB, D), wg)
    exp_counts = stats[0, :E].astype(jnp.int32)
    l_aux = stats[1, 0]
    idx4 = t2s3.reshape(NW, NCH, CH)
    g4 = gtab.reshape(NW, NCH, CH, GW)
    disp, sg = _make_dispatch()(x, idx4, g4)
    eo = _experts(exp_counts, disp, sg, w1, b1.reshape(E, 1, DFF),
                  w2, b2.reshape(E, 1, D))
    out = _make_combine()(eo, idx4)
    return out.reshape(hidden_states.shape), l_aux, exp_counts


# speedup vs baseline: 1.0902x; 1.0037x over previous
import functools

import jax
import jax.numpy as jnp
from jax import lax
from jax.experimental import pallas as pl
from jax.experimental.pallas import tpu as pltpu
from jax.experimental.pallas import tpu_sc as plsc

S = 4096
D = 1024
E = 64
DFF = 2048
CAP = 64
TB = 512
NB = S // TB
NSLOT = E * CAP
TRASH = NSLOT
NDISP = NSLOT + CAP
GW = 128
NW = 32
TPW = S // NW
NCH = 8
NBUF = 4
CH = TPW // NCH


def _routing_body(x_ref, wg_ref, t2s_ref, gate_ref, stats_ref, acc_ref):
    step = pl.program_id(0)

    @pl.when(step == 0)
    def _init():
        acc_ref[...] = jnp.zeros_like(acc_ref)

    x = x_ref[0]
    logits = jnp.dot(x, wg_ref[...], preferred_element_type=jnp.float32)
    m = jnp.max(logits, axis=1, keepdims=True)
    ex = jnp.exp(logits - m)
    gates = ex / jnp.sum(ex, axis=1, keepdims=True)

    iota_e = lax.broadcasted_iota(jnp.int32, (TB, E), 1)
    eq = logits == m
    idxe = jnp.min(jnp.where(eq, iota_e, E), axis=1)
    maskf = (iota_e == idxe[:, None]).astype(jnp.float32)

    ti = lax.broadcasted_iota(jnp.int32, (TB, TB), 0)
    tj = lax.broadcasted_iota(jnp.int32, (TB, TB), 1)
    tri = (tj < ti).astype(jnp.float32)
    carry = acc_ref[0:1, 0:E]
    pos = jnp.dot(tri, maskf, preferred_element_type=jnp.float32) + carry
    keptf = maskf * (pos < CAP).astype(jnp.float32)

    gate_s = jnp.sum(gates * keptf, axis=1)
    pos_s = jnp.sum(pos * keptf, axis=1)
    has = jnp.sum(keptf, axis=1) > 0.0
    slot = jnp.where(has, idxe * CAP + pos_s.astype(jnp.int32), TRASH)
    t2s_ref[0, 0, :] = slot
    gate_ref[...] = jnp.broadcast_to(gate_s[:, None], (TB, GW))

    acc_ref[0:1, 0:E] = carry + jnp.sum(maskf, axis=0, keepdims=True)
    acc_ref[1:2, 0:E] = acc_ref[1:2, 0:E] + jnp.sum(gates, axis=0, keepdims=True)

    @pl.when(step == NB - 1)
    def _fin():
        counts = acc_ref[0:1, 0:E]
        mes = acc_ref[1:2, 0:E]
        l_aux = jnp.sum((mes / S) * (counts / S)) * E
        stats_ref[0:1, 0:E] = counts
        stats_ref[1:2, 0:1] = jnp.reshape(l_aux, (1, 1))


_routing = pl.pallas_call(
    _routing_body,
    grid=(NB,),
    in_specs=[
        pl.BlockSpec((1, TB, D), lambda i: (i, 0, 0)),
        pl.BlockSpec((D, E), lambda i: (0, 0)),
    ],
    out_specs=[
        pl.BlockSpec((1, 1, TB), lambda i: (i, 0, 0)),
        pl.BlockSpec((TB, GW), lambda i: (i, 0)),
        pl.BlockSpec((8, 128), lambda i: (0, 0)),
    ],
    out_shape=[
        jax.ShapeDtypeStruct((NB, 1, TB), jnp.int32),
        jax.ShapeDtypeStruct((S, GW), jnp.float32),
        jax.ShapeDtypeStruct((8, 128), jnp.float32),
    ],
    scratch_shapes=[pltpu.VMEM((8, 128), jnp.float32)],
)


@functools.cache
def _make_dispatch():
    mesh = plsc.VectorSubcoreMesh(core_axis_name="c", subcore_axis_name="s")

    @functools.partial(
        pl.kernel,
        out_type=[
            jax.ShapeDtypeStruct((NDISP, D), jnp.float32),
            jax.ShapeDtypeStruct((NDISP, GW), jnp.float32),
        ],
        mesh=mesh,
        scratch_types=[
            pltpu.VMEM((NCH, CH), jnp.int32),
            [pltpu.VMEM((CH, D), jnp.float32) for _ in range(NBUF)],
            pltpu.VMEM((NCH, CH, GW), jnp.float32),
            [pltpu.SemaphoreType.DMA for _ in range(NBUF)],
            [pltpu.SemaphoreType.DMA for _ in range(NBUF)],
            pltpu.SemaphoreType.DMA,
        ],
    )
    def _dispatch(x_hbm, idx4_hbm, g4_hbm, disp_hbm, sg_hbm,
                  idx_v, bufs, g_all, lsems, ssems, gs):
        wid = lax.axis_index("s") * 2 + lax.axis_index("c")
        base = wid * TPW
        pltpu.sync_copy(idx4_hbm.at[wid], idx_v)
        pltpu.sync_copy(g4_hbm.at[wid], g_all)
        gate_dmas = [
            pltpu.async_copy(g_all.at[q], sg_hbm.at[idx_v.at[q]], gs)
            for q in range(NCH)
        ]
        loads = [None] * NCH
        scats = [None] * NCH

        def load(q):
            b = q % NBUF
            loads[q] = pltpu.async_copy(
                x_hbm.at[pl.ds(base + q * CH, CH)], bufs[b], lsems[b])

        for q in range(NBUF):
            load(q)
        for q in range(NCH):
            b = q % NBUF
            loads[q].wait()
            scats[q] = pltpu.async_copy(bufs[b], disp_hbm.at[idx_v.at[q]],
                                        ssems[b])
            r = q - (NBUF - 1)
            if r >= 0 and r + NBUF < NCH:
                scats[r].wait()
                load(r + NBUF)
        for q in range(NCH - NBUF, NCH):
            scats[q].wait()
        for dma in gate_dmas:
            dma.wait()

    return _dispatch


def _experts_body(cnt_ref, disp_ref, sg_ref, w1_ref, b1_ref, w2_ref, b2_ref,
                  out_ref):
    e = pl.program_id(0)

    @pl.when(e < E)
    def _mlp():
        cnt = cnt_ref[jnp.minimum(e, E - 1)]
        rows = lax.broadcasted_iota(jnp.int32, (CAP, D), 0)
        keep = rows < cnt
        xb = jnp.where(keep, disp_ref[...], 0.0)
        h = jnp.dot(xb, w1_ref[0], preferred_element_type=jnp.float32)
        h = jnp.maximum(h + b1_ref[0], 0.0)
        y = jnp.dot(h, w2_ref[0], preferred_element_type=jnp.float32)
        y = y + b2_ref[0]
        sg = sg_ref[:, 0:1]
        out_ref[...] = jnp.where(keep, y * sg, 0.0)

    @pl.when(e >= E)
    def _trash():
        out_ref[...] = jnp.zeros_like(out_ref)


_experts = pl.pallas_call(
    _experts_body,
    grid=(E + 1,),
    in_specs=[
        pl.BlockSpec(memory_space=pltpu.SMEM),
        pl.BlockSpec((CAP, D), lambda e: (e, 0)),
        pl.BlockSpec((CAP, GW), lambda e: (e, 0)),
        pl.BlockSpec((1, D, DFF), lambda e: (jnp.minimum(e, E - 1), 0, 0)),
        pl.BlockSpec((1, 1, DFF), lambda e: (jnp.minimum(e, E - 1), 0, 0)),
        pl.BlockSpec((1, DFF, D), lambda e: (jnp.minimum(e, E - 1), 0, 0)),
        pl.BlockSpec((1, 1, D), lambda e: (jnp.minimum(e, E - 1), 0, 0)),
    ],
    out_specs=pl.BlockSpec((CAP, D), lambda e: (e, 0)),
    out_shape=jax.ShapeDtypeStruct((NDISP, D), jnp.float32),
)


@functools.cache
def _make_combine():
    mesh = plsc.VectorSubcoreMesh(core_axis_name="c", subcore_axis_name="s")

    @functools.partial(
        pl.kernel,
        out_type=jax.ShapeDtypeStruct((S, D), jnp.float32),
        mesh=mesh,
        scratch_types=[
            pltpu.VMEM((NCH, CH), jnp.int32),
            [pltpu.VMEM((CH, D), jnp.float32) for _ in range(NBUF)],
            [pltpu.SemaphoreType.DMA for _ in range(NBUF)],
            [pltpu.SemaphoreType.DMA for _ in range(NBUF)],
        ],
    )
    def _combine(eo_hbm, idx4_hbm, out_hbm, idx_v, bufs, gsems, ssems):
        wid = lax.axis_index("s") * 2 + lax.axis_index("c")
        base = wid * TPW
        pltpu.sync_copy(idx4_hbm.at[wid], idx_v)
        gets = [None] * NCH
        puts = [None] * NCH

        def get(q):
            b = q % NBUF
            gets[q] = pltpu.async_copy(eo_hbm.at[idx_v.at[q]], bufs[b],
                                       gsems[b])

        for q in range(NBUF):
            get(q)
        for q in range(NCH):
            b = q % NBUF
            gets[q].wait()
            puts[q] = pltpu.async_copy(
                bufs[b], out_hbm.at[pl.ds(base + q * CH, CH)], ssems[b])
            r = q - (NBUF - 1)
            if r >= 0 and r + NBUF < NCH:
                puts[r].wait()
                get(r + NBUF)
        for q in range(NCH - NBUF, NCH):
            puts[q].wait()

    return _combine


def kernel(hidden_states, wg, w1, b1, w2, b2):
    x = hidden_states.reshape(S, D)
    t2s3, gtab, stats = _routing(x.reshape(NB, TB, D), wg)
    exp_counts = stats[0, :E].astype(jnp.int32)
    l_aux = stats[1, 0]
    idx4 = t2s3.reshape(NW, NCH, CH)
    g4 = gtab.reshape(NW, NCH, CH, GW)
    disp, sg = _make_dispatch()(x, idx4, g4)
    eo = _experts(exp_counts, disp, sg, w1, b1.reshape(E, 1, DFF),
                  w2, b2.reshape(E, 1, D))
    out = _make_combine()(eo, idx4)
    return out.reshape(hidden_states.shape), l_aux, exp_counts


# speedup vs baseline: 1.0949x; 1.0043x over previous
import functools

import jax
import jax.numpy as jnp
from jax import lax
from jax.experimental import pallas as pl
from jax.experimental.pallas import tpu as pltpu
from jax.experimental.pallas import tpu_sc as plsc

S = 4096
D = 1024
E = 64
DFF = 2048
CAP = 64
TB = 512
NB = S // TB
NSLOT = E * CAP
TRASH = NSLOT
NDISP = NSLOT + CAP
GW = 128
NW = 32
TPW = S // NW
NCH = 4
NBUF = 3
CH = TPW // NCH


def _routing_body(x_ref, wg_ref, t2s_ref, gate_ref, stats_ref, acc_ref):
    step = pl.program_id(0)

    @pl.when(step == 0)
    def _init():
        acc_ref[...] = jnp.zeros_like(acc_ref)

    x = x_ref[0]
    logits = jnp.dot(x, wg_ref[...], preferred_element_type=jnp.float32)
    m = jnp.max(logits, axis=1, keepdims=True)
    ex = jnp.exp(logits - m)
    gates = ex / jnp.sum(ex, axis=1, keepdims=True)

    iota_e = lax.broadcasted_iota(jnp.int32, (TB, E), 1)
    eq = logits == m
    idxe = jnp.min(jnp.where(eq, iota_e, E), axis=1)
    maskf = (iota_e == idxe[:, None]).astype(jnp.float32)

    ti = lax.broadcasted_iota(jnp.int32, (TB, TB), 0)
    tj = lax.broadcasted_iota(jnp.int32, (TB, TB), 1)
    tri = (tj < ti).astype(jnp.float32)
    carry = acc_ref[0:1, 0:E]
    pos = jnp.dot(tri, maskf, preferred_element_type=jnp.float32) + carry
    keptf = maskf * (pos < CAP).astype(jnp.float32)

    gate_s = jnp.sum(gates * keptf, axis=1)
    pos_s = jnp.sum(pos * keptf, axis=1)
    has = jnp.sum(keptf, axis=1) > 0.0
    slot = jnp.where(has, idxe * CAP + pos_s.astype(jnp.int32), TRASH)
    t2s_ref[0, 0, :] = slot
    gate_ref[...] = jnp.broadcast_to(gate_s[:, None], (TB, GW))

    acc_ref[0:1, 0:E] = carry + jnp.sum(maskf, axis=0, keepdims=True)
    acc_ref[1:2, 0:E] = acc_ref[1:2, 0:E] + jnp.sum(gates, axis=0, keepdims=True)

    @pl.when(step == NB - 1)
    def _fin():
        counts = acc_ref[0:1, 0:E]
        mes = acc_ref[1:2, 0:E]
        l_aux = jnp.sum((mes / S) * (counts / S)) * E
        stats_ref[0:1, 0:E] = counts
        stats_ref[1:2, 0:1] = jnp.reshape(l_aux, (1, 1))


_routing = pl.pallas_call(
    _routing_body,
    grid=(NB,),
    in_specs=[
        pl.BlockSpec((1, TB, D), lambda i: (i, 0, 0)),
        pl.BlockSpec((D, E), lambda i: (0, 0)),
    ],
    out_specs=[
        pl.BlockSpec((1, 1, TB), lambda i: (i, 0, 0)),
        pl.BlockSpec((TB, GW), lambda i: (i, 0)),
        pl.BlockSpec((8, 128), lambda i: (0, 0)),
    ],
    out_shape=[
        jax.ShapeDtypeStruct((NB, 1, TB), jnp.int32),
        jax.ShapeDtypeStruct((S, GW), jnp.float32),
        jax.ShapeDtypeStruct((8, 128), jnp.float32),
    ],
    scratch_shapes=[pltpu.VMEM((8, 128), jnp.float32)],
)


@functools.cache
def _make_dispatch():
    mesh = plsc.VectorSubcoreMesh(core_axis_name="c", subcore_axis_name="s")

    @functools.partial(
        pl.kernel,
        out_type=[
            jax.ShapeDtypeStruct((NDISP, D), jnp.float32),
            jax.ShapeDtypeStruct((NDISP, GW), jnp.float32),
        ],
        mesh=mesh,
        scratch_types=[
            pltpu.VMEM((NCH, CH), jnp.int32),
            [pltpu.VMEM((CH, D), jnp.float32) for _ in range(NBUF)],
            pltpu.VMEM((NCH, CH, GW), jnp.float32),
            [pltpu.SemaphoreType.DMA for _ in range(NBUF)],
            [pltpu.SemaphoreType.DMA for _ in range(NBUF)],
            pltpu.SemaphoreType.DMA,
        ],
    )
    def _dispatch(x_hbm, idx4_hbm, g4_hbm, disp_hbm, sg_hbm,
                  idx_v, bufs, g_all, lsems, ssems, gs):
        wid = lax.axis_index("s") * 2 + lax.axis_index("c")
        base = wid * TPW
        pltpu.sync_copy(idx4_hbm.at[wid], idx_v)
        pltpu.sync_copy(g4_hbm.at[wid], g_all)
        gate_dmas = [
            pltpu.async_copy(g_all.at[q], sg_hbm.at[idx_v.at[q]], gs)
            for q in range(NCH)
        ]
        loads = [None] * NCH
        scats = [None] * NCH

        def load(q):
            b = q % NBUF
            loads[q] = pltpu.async_copy(
                x_hbm.at[pl.ds(base + q * CH, CH)], bufs[b], lsems[b])

        for q in range(NBUF):
            load(q)
        for q in range(NCH):
            b = q % NBUF
            loads[q].wait()
            scats[q] = pltpu.async_copy(bufs[b], disp_hbm.at[idx_v.at[q]],
                                        ssems[b])
            r = q - (NBUF - 1)
            if r >= 0 and r + NBUF < NCH:
                scats[r].wait()
                load(r + NBUF)
        for q in range(NCH - NBUF, NCH):
            scats[q].wait()
        for dma in gate_dmas:
            dma.wait()

    return _dispatch


DH = DFF // 2


def _experts_body(cnt_ref, disp_ref, sg_ref, w1_ref, b1_ref, w2_ref, b2_ref,
                  out_ref, acc_ref):
    e = pl.program_id(0)
    j = pl.program_id(1)

    @pl.when(e < E)
    def _mlp():
        cnt = cnt_ref[jnp.minimum(e, E - 1)]
        rows = lax.broadcasted_iota(jnp.int32, (CAP, D), 0)
        keep = rows < cnt
        xb = jnp.where(keep, disp_ref[...], 0.0)
        h = jnp.dot(xb, w1_ref[0], preferred_element_type=jnp.float32)
        h = jnp.maximum(h + b1_ref[0], 0.0)
        y = jnp.dot(h, w2_ref[0], preferred_element_type=jnp.float32)

        @pl.when(j == 0)
        def _():
            acc_ref[...] = y

        @pl.when(j == 1)
        def _():
            full = acc_ref[...] + y + b2_ref[0]
            sg = sg_ref[:, 0:1]
            out_ref[...] = jnp.where(keep, full * sg, 0.0)

    @pl.when(jnp.logical_and(e >= E, j == 1))
    def _trash():
        out_ref[...] = jnp.zeros_like(out_ref)


_experts = pl.pallas_call(
    _experts_body,
    grid=(E + 1, 2),
    in_specs=[
        pl.BlockSpec(memory_space=pltpu.SMEM),
        pl.BlockSpec((CAP, D), lambda e, j: (e, 0)),
        pl.BlockSpec((CAP, GW), lambda e, j: (e, 0)),
        pl.BlockSpec((1, D, DH), lambda e, j: (jnp.minimum(e, E - 1), 0, j)),
        pl.BlockSpec((1, 1, DH), lambda e, j: (jnp.minimum(e, E - 1), 0, j)),
        pl.BlockSpec((1, DH, D), lambda e, j: (jnp.minimum(e, E - 1), j, 0)),
        pl.BlockSpec((1, 1, D), lambda e, j: (jnp.minimum(e, E - 1), 0, 0)),
    ],
    out_specs=pl.BlockSpec((CAP, D), lambda e, j: (e, 0)),
    out_shape=jax.ShapeDtypeStruct((NDISP, D), jnp.float32),
    scratch_shapes=[pltpu.VMEM((CAP, D), jnp.float32)],
)


@functools.cache
def _make_combine():
    mesh = plsc.VectorSubcoreMesh(core_axis_name="c", subcore_axis_name="s")

    @functools.partial(
        pl.kernel,
        out_type=jax.ShapeDtypeStruct((S, D), jnp.float32),
        mesh=mesh,
        scratch_types=[
            pltpu.VMEM((NCH, CH), jnp.int32),
            [pltpu.VMEM((CH, D), jnp.float32) for _ in range(NBUF)],
            [pltpu.SemaphoreType.DMA for _ in range(NBUF)],
            [pltpu.SemaphoreType.DMA for _ in range(NBUF)],
        ],
    )
    def _combine(eo_hbm, idx4_hbm, out_hbm, idx_v, bufs, gsems, ssems):
        wid = lax.axis_index("s") * 2 + lax.axis_index("c")
        base = wid * TPW
        pltpu.sync_copy(idx4_hbm.at[wid], idx_v)
        gets = [None] * NCH
        puts = [None] * NCH

        def get(q):
            b = q % NBUF
            gets[q] = pltpu.async_copy(eo_hbm.at[idx_v.at[q]], bufs[b],
                                       gsems[b])

        for q in range(NBUF):
            get(q)
        for q in range(NCH):
            b = q % NBUF
            gets[q].wait()
            puts[q] = pltpu.async_copy(
                bufs[b], out_hbm.at[pl.ds(base + q * CH, CH)], ssems[b])
            r = q - (NBUF - 1)
            if r >= 0 and r + NBUF < NCH:
                puts[r].wait()
                get(r + NBUF)
        for q in range(NCH - NBUF, NCH):
            puts[q].wait()

    return _combine


def kernel(hidden_states, wg, w1, b1, w2, b2):
    x = hidden_states.reshape(S, D)
    t2s3, gtab, stats = _routing(x.reshape(NB, TB, D), wg)
    exp_counts = stats[0, :E].astype(jnp.int32)
    l_aux = stats[1, 0]
    idx4 = t2s3.reshape(NW, NCH, CH)
    g4 = gtab.reshape(NW, NCH, CH, GW)
    disp, sg = _make_dispatch()(x, idx4, g4)
    eo = _experts(exp_counts, disp, sg, w1, b1.reshape(E, 1, DFF),
                  w2, b2.reshape(E, 1, D))
    out = _make_combine()(eo, idx4)
    return out.reshape(hidden_states.shape), l_aux, exp_counts


# speedup vs baseline: 1.0949x; 1.0001x over previous
import functools

import jax
import jax.numpy as jnp
from jax import lax
from jax.experimental import pallas as pl
from jax.experimental.pallas import tpu as pltpu
from jax.experimental.pallas import tpu_sc as plsc

S = 4096
D = 1024
E = 64
DFF = 2048
CAP = 64
TB = 512
NB = S // TB
NSLOT = E * CAP
TRASH = NSLOT
NDISP = NSLOT + CAP
GW = 128
NW = 32
TPW = S // NW
NCH = 4
NBUF = 3
CH = TPW // NCH


def _routing_body(x_ref, wg_ref, t2s_ref, gate_ref, stats_ref, acc_ref):
    step = pl.program_id(0)

    @pl.when(step == 0)
    def _init():
        acc_ref[...] = jnp.zeros_like(acc_ref)

    x = x_ref[0]
    logits = jnp.dot(x, wg_ref[...], preferred_element_type=jnp.float32)
    m = jnp.max(logits, axis=1, keepdims=True)
    ex = jnp.exp(logits - m)
    gates = ex / jnp.sum(ex, axis=1, keepdims=True)

    iota_e = lax.broadcasted_iota(jnp.int32, (TB, E), 1)
    eq = logits == m
    idxe = jnp.min(jnp.where(eq, iota_e, E), axis=1)
    maskf = (iota_e == idxe[:, None]).astype(jnp.float32)

    ti = lax.broadcasted_iota(jnp.int32, (TB, TB), 0)
    tj = lax.broadcasted_iota(jnp.int32, (TB, TB), 1)
    tri = (tj < ti).astype(jnp.float32)
    carry = acc_ref[0:1, 0:E]
    pos = jnp.dot(tri, maskf, preferred_element_type=jnp.float32) + carry
    keptf = maskf * (pos < CAP).astype(jnp.float32)

    gate_s = jnp.sum(gates * keptf, axis=1)
    pos_s = jnp.sum(pos * keptf, axis=1)
    has = jnp.sum(keptf, axis=1) > 0.0
    slot = jnp.where(has, idxe * CAP + pos_s.astype(jnp.int32), TRASH)
    t2s_ref[0, 0, :] = slot
    gate_ref[...] = jnp.broadcast_to(gate_s[:, None], (TB, GW))

    acc_ref[0:1, 0:E] = carry + jnp.sum(maskf, axis=0, keepdims=True)
    acc_ref[1:2, 0:E] = acc_ref[1:2, 0:E] + jnp.sum(gates, axis=0, keepdims=True)

    @pl.when(step == NB - 1)
    def _fin():
        counts = acc_ref[0:1, 0:E]
        mes = acc_ref[1:2, 0:E]
        l_aux = jnp.sum((mes / S) * (counts / S)) * E
        stats_ref[0:1, 0:E] = counts
        stats_ref[1:2, 0:1] = jnp.reshape(l_aux, (1, 1))


_routing = pl.pallas_call(
    _routing_body,
    grid=(NB,),
    in_specs=[
        pl.BlockSpec((1, TB, D), lambda i: (i, 0, 0)),
        pl.BlockSpec((D, E), lambda i: (0, 0)),
    ],
    out_specs=[
        pl.BlockSpec((1, 1, TB), lambda i: (i, 0, 0)),
        pl.BlockSpec((TB, GW), lambda i: (i, 0)),
        pl.BlockSpec((8, 128), lambda i: (0, 0)),
    ],
    out_shape=[
        jax.ShapeDtypeStruct((NB, 1, TB), jnp.int32),
        jax.ShapeDtypeStruct((S, GW), jnp.float32),
        jax.ShapeDtypeStruct((8, 128), jnp.float32),
    ],
    scratch_shapes=[pltpu.VMEM((8, 128), jnp.float32)],
)


@functools.cache
def _make_dispatch():
    mesh = plsc.VectorSubcoreMesh(core_axis_name="c", subcore_axis_name="s")

    @functools.partial(
        pl.kernel,
        out_type=[
            jax.ShapeDtypeStruct((NDISP, D), jnp.float32),
            jax.ShapeDtypeStruct((NDISP, GW), jnp.float32),
        ],
        mesh=mesh,
        scratch_types=[
            pltpu.VMEM((NCH, CH), jnp.int32),
            [pltpu.VMEM((CH, D), jnp.float32) for _ in range(NBUF)],
            pltpu.VMEM((NCH, CH, GW), jnp.float32),
            [pltpu.SemaphoreType.DMA for _ in range(NBUF)],
            [pltpu.SemaphoreType.DMA for _ in range(NBUF)],
            pltpu.SemaphoreType.DMA,
        ],
    )
    def _dispatch(x_hbm, idx4_hbm, g4_hbm, disp_hbm, sg_hbm,
                  idx_v, bufs, g_all, lsems, ssems, gs):
        wid = lax.axis_index("s") * 2 + lax.axis_index("c")
        base = wid * TPW
        pltpu.sync_copy(idx4_hbm.at[wid], idx_v)
        pltpu.sync_copy(g4_hbm.at[wid], g_all)
        gate_dmas = [
            pltpu.async_copy(g_all.at[q], sg_hbm.at[idx_v.at[q]], gs)
            for q in range(NCH)
        ]
        loads = [None] * NCH
        scats = [None] * NCH

        def load(q):
            b = q % NBUF
            loads[q] = pltpu.async_copy(
                x_hbm.at[pl.ds(base + q * CH, CH)], bufs[b], lsems[b])

        for q in range(NBUF):
            load(q)
        for q in range(NCH):
            b = q % NBUF
            loads[q].wait()
            scats[q] = pltpu.async_copy(bufs[b], disp_hbm.at[idx_v.at[q]],
                                        ssems[b])
            r = q - (NBUF - 1)
            if r >= 0 and r + NBUF < NCH:
                scats[r].wait()
                load(r + NBUF)
        for q in range(NCH - NBUF, NCH):
            scats[q].wait()
        for dma in gate_dmas:
            dma.wait()

    return _dispatch


def _experts_body(cnt_ref, disp_ref, sg_ref, w1_ref, b1_ref, w2_ref, b2_ref,
                  out_ref):
    e = pl.program_id(0)

    @pl.when(e < E)
    def _mlp():
        cnt = cnt_ref[jnp.minimum(e, E - 1)]
        rows = lax.broadcasted_iota(jnp.int32, (CAP, D), 0)
        keep = rows < cnt
        xb = jnp.where(keep, disp_ref[...], 0.0)
        h = jnp.dot(xb, w1_ref[0], preferred_element_type=jnp.float32)
        h = jnp.maximum(h + b1_ref[0], 0.0)
        y = jnp.dot(h, w2_ref[0], preferred_element_type=jnp.float32)
        full = y + b2_ref[0]
        sg = sg_ref[:, 0:1]
        out_ref[...] = jnp.where(keep, full * sg, 0.0)

    @pl.when(e >= E)
    def _trash():
        out_ref[...] = jnp.zeros_like(out_ref)


_experts = pl.pallas_call(
    _experts_body,
    grid=(E + 1,),
    in_specs=[
        pl.BlockSpec(memory_space=pltpu.SMEM),
        pl.BlockSpec((CAP, D), lambda e: (e, 0)),
        pl.BlockSpec((CAP, GW), lambda e: (e, 0)),
        pl.BlockSpec((1, D, DFF), lambda e: (jnp.minimum(e, E - 1), 0, 0)),
        pl.BlockSpec((1, 1, DFF), lambda e: (jnp.minimum(e, E - 1), 0, 0)),
        pl.BlockSpec((1, DFF, D), lambda e: (jnp.minimum(e, E - 1), 0, 0)),
        pl.BlockSpec((1, 1, D), lambda e: (jnp.minimum(e, E - 1), 0, 0)),
    ],
    out_specs=pl.BlockSpec((CAP, D), lambda e: (e, 0)),
    out_shape=jax.ShapeDtypeStruct((NDISP, D), jnp.float32),
)


@functools.cache
def _make_combine():
    mesh = plsc.VectorSubcoreMesh(core_axis_name="c", subcore_axis_name="s")

    @functools.partial(
        pl.kernel,
        out_type=jax.ShapeDtypeStruct((S, D), jnp.float32),
        mesh=mesh,
        scratch_types=[
            pltpu.VMEM((NCH, CH), jnp.int32),
            [pltpu.VMEM((CH, D), jnp.float32) for _ in range(NBUF)],
            [pltpu.SemaphoreType.DMA for _ in range(NBUF)],
            [pltpu.SemaphoreType.DMA for _ in range(NBUF)],
        ],
    )
    def _combine(eo_hbm, idx4_hbm, out_hbm, idx_v, bufs, gsems, ssems):
        wid = lax.axis_index("s") * 2 + lax.axis_index("c")
        base = wid * TPW
        pltpu.sync_copy(idx4_hbm.at[wid], idx_v)
        gets = [None] * NCH
        puts = [None] * NCH

        def get(q):
            b = q % NBUF
            gets[q] = pltpu.async_copy(eo_hbm.at[idx_v.at[q]], bufs[b],
                                       gsems[b])

        for q in range(NBUF):
            get(q)
        for q in range(NCH):
            b = q % NBUF
            gets[q].wait()
            puts[q] = pltpu.async_copy(
                bufs[b], out_hbm.at[pl.ds(base + q * CH, CH)], ssems[b])
            r = q - (NBUF - 1)
            if r >= 0 and r + NBUF < NCH:
                puts[r].wait()
                get(r + NBUF)
        for q in range(NCH - NBUF, NCH):
            puts[q].wait()

    return _combine


def kernel(hidden_states, wg, w1, b1, w2, b2):
    x = hidden_states.reshape(S, D)
    t2s3, gtab, stats = _routing(x.reshape(NB, TB, D), wg)
    exp_counts = stats[0, :E].astype(jnp.int32)
    l_aux = stats[1, 0]
    idx4 = t2s3.reshape(NW, NCH, CH)
    g4 = gtab.reshape(NW, NCH, CH, GW)
    disp, sg = _make_dispatch()(x, idx4, g4)
    eo = _experts(exp_counts, disp, sg, w1, b1.reshape(E, 1, DFF),
                  w2, b2.reshape(E, 1, D))
    out = _make_combine()(eo, idx4)
    return out.reshape(hidden_states.shape), l_aux, exp_counts
